# baseline (speedup 1.0000x reference)
import functools

import jax
import jax.numpy as jnp
from jax import lax
from jax.experimental import pallas as pl
from jax.experimental.pallas import tpu as pltpu
from jax.experimental.pallas import tpu_sc as plsc

N_NODES = 10000
D_FEAT = 256
N_GRAPHS = 64

NC = 2
NS = 16
NW = NC * NS

CHUNK = 128
NFULL = N_NODES // CHUNK
TAIL = N_NODES - NFULL * CHUNK
KMAX = (NFULL + NW - 1) // NW
IDS_PAD = 10240


def _sc_segment_sum(x, batch_i32):
  mesh = plsc.VectorSubcoreMesh(core_axis_name="c", subcore_axis_name="s")

  @functools.partial(
      pl.kernel,
      out_type=jax.ShapeDtypeStruct((NW, N_GRAPHS, D_FEAT), jnp.float32),
      mesh=mesh,
      scratch_types=[
          pltpu.VMEM((KMAX * CHUNK + TAIL, D_FEAT), jnp.float32),
          pltpu.VMEM((KMAX * CHUNK + TAIL,), jnp.int32),
          pltpu.VMEM((N_GRAPHS, D_FEAT), jnp.float32),
          pltpu.SemaphoreType.DMA,
          pltpu.SemaphoreType.DMA,
          pltpu.SemaphoreType.DMA,
          pltpu.SemaphoreType.DMA,
      ],
  )
  def k(x_hbm, ids_hbm, sums_hbm, rows_v, idx_v, acc_v, sem0, sem1, sem2, semt):
    c = lax.axis_index("c")
    s = lax.axis_index("s")
    wid = s * NC + c
    sems = [sem0, sem1, sem2]

    for kk in range(KMAX):
      ci = wid + NW * kk

      @pl.when(ci < NFULL)
      def _():
        base = ci * CHUNK
        dst_r = rows_v.at[pl.ds(kk * CHUNK, CHUNK)]
        dst_i = idx_v.at[pl.ds(kk * CHUNK, CHUNK)]
        pltpu.async_copy(x_hbm.at[pl.ds(base, CHUNK)], dst_r, sems[kk])
        pltpu.async_copy(ids_hbm.at[pl.ds(base, CHUNK)], dst_i, sems[kk])

    @pl.when(wid == NW - 1)
    def _():
      base = NFULL * CHUNK
      row0 = 2 * CHUNK
      pltpu.async_copy(
          x_hbm.at[pl.ds(base, TAIL)], rows_v.at[pl.ds(row0, TAIL)], semt)
      pltpu.async_copy(
          ids_hbm.at[pl.ds(base, TAIL)], idx_v.at[pl.ds(row0, TAIL)], semt)

    zero = jnp.zeros((16,), jnp.float32)

    def zrow(r, carry):
      for j in range(D_FEAT // 16):
        acc_v[r, pl.ds(16 * j, 16)] = zero
      return carry

    lax.fori_loop(0, N_GRAPHS, zrow, 0)

    for kk in range(KMAX):
      ci = wid + NW * kk

      @pl.when(ci < NFULL)
      def _():
        dst_r = rows_v.at[pl.ds(kk * CHUNK, CHUNK)]
        dst_i = idx_v.at[pl.ds(kk * CHUNK, CHUNK)]
        pltpu.make_async_copy(x_hbm.at[pl.ds(0, CHUNK)], dst_r, sems[kk]).wait()
        pltpu.make_async_copy(ids_hbm.at[pl.ds(0, CHUNK)], dst_i, sems[kk]).wait()

    @pl.when(wid == NW - 1)
    def _():
      row0 = 2 * CHUNK
      pltpu.make_async_copy(
          x_hbm.at[pl.ds(0, TAIL)], rows_v.at[pl.ds(row0, TAIL)], semt).wait()
      pltpu.make_async_copy(
          ids_hbm.at[pl.ds(0, TAIL)], idx_v.at[pl.ds(row0, TAIL)], semt).wait()

    nk = (NFULL - 1 - wid) // NW + 1
    ngroups = nk * (CHUNK // 16) + jnp.where(wid == NW - 1, 1, 0)

    def rowgroup(t, carry):
      gvec = idx_v[pl.ds(16 * t, 16)]
      g0 = gvec[0]

      @pl.when(g0 == gvec[15])
      def _():
        for j in range(D_FEAT // 16):
          sl = pl.ds(16 * j, 16)
          v = [rows_v[16 * t + l, sl] for l in range(16)]
          while len(v) > 1:
            v = [a + b for a, b in zip(v[::2], v[1::2])]
          acc_v[g0, sl] = acc_v[g0, sl] + v[0]

      @pl.when(g0 != gvec[15])
      def _():
        for l in range(16):
          g = gvec[l]
          r = 16 * t + l
          for j in range(D_FEAT // 16):
            sl = pl.ds(16 * j, 16)
            acc_v[g, sl] = acc_v[g, sl] + rows_v[r, sl]

      return carry

    lax.fori_loop(0, ngroups, rowgroup, 0)

    pltpu.sync_copy(acc_v, sums_hbm.at[wid])

  return k(x, batch_i32)


def _tc_mlp(sums32, ids_pad, u, W1, b1, W2, b2):
  def body(sums_ref, ids_ref, u_ref, W1_ref, b1_ref, W2_ref, b2_ref, o_ref):
    sums = jnp.sum(sums_ref[...], axis=0)
    gid = lax.broadcasted_iota(jnp.int32, (N_GRAPHS, 1), 0)
    eq = (ids_ref[...] == gid).astype(jnp.float32)
    cnt = jnp.sum(eq, axis=1, keepdims=True)
    mean = sums / jnp.maximum(cnt, 1.0)
    d_g = u_ref.shape[1]
    z = (
        jnp.dot(u_ref[...], W1_ref[0:d_g, :], preferred_element_type=jnp.float32)
        + jnp.dot(mean, W1_ref[d_g:, :], preferred_element_type=jnp.float32)
        + b1_ref[...]
    )
    h = jnp.where(z > 0, z, jnp.exp(jnp.minimum(z, 0.0)) - 1.0)
    o_ref[...] = (
        jnp.dot(h, W2_ref[...], preferred_element_type=jnp.float32) + b2_ref[...]
    )

  return pl.pallas_call(
      body,
      out_shape=jax.ShapeDtypeStruct((u.shape[0], W2.shape[1]), jnp.float32),
  )(sums32, ids_pad, u, W1, b1.reshape(1, -1), W2, b2.reshape(1, -1))


def kernel(x, edge_index, edge_attr, u, batch, W1, b1, W2, b2):
  del edge_index, edge_attr
  batch_i32 = batch.astype(jnp.int32)
  ids_pad = jnp.full((1, IDS_PAD), N_GRAPHS, jnp.int32)
  ids_pad = lax.dynamic_update_slice(ids_pad, batch_i32.reshape(1, -1), (0, 0))
  sums32 = _sc_segment_sum(x, batch_i32)
  return _tc_mlp(sums32, ids_pad, u, W1, b1, W2, b2)
